# baseline (speedup 1.0000x reference)
import functools

import jax
import jax.numpy as jnp
from jax import lax
from jax.experimental import pallas as pl
from jax.experimental.pallas import tpu as pltpu
from jax.experimental.pallas import tpu_sc as plsc

B, N, H = 16, 128, 32
MHD, EF = 5, 3
NER = 513
NSP = 512
TBL_ROWS = MHD * NER + NSP
HP = H // 2
NC, NS, L = 2, 16, 16
NW = NC * NS
ROWS = B * N
ROWS_PER_W = ROWS // NW
INW = N * 16


def _prep_body(ew_ref, w3_ref, sw_ref, out_ref):
    ew = ew_ref[...]
    parts = [jnp.dot(ew, w3_ref[m], preferred_element_type=jnp.float32)
             for m in range(MHD)]
    parts.append(sw_ref[...])
    out_ref[...] = jnp.concatenate(parts, axis=0)


def _prep_table(edge_weight, w3, spatial_weight):
    return pl.pallas_call(
        _prep_body,
        out_shape=jax.ShapeDtypeStruct((TBL_ROWS, H), jnp.float32),
    )(edge_weight, w3, spatial_weight)


def _row_compute(in_v, r_v, tbl_v, lane, offc, pa, pb, pc, p15, ioN):
    def pair_body(j, carry2):
        iv = in_v[pl.ds(j * 16, 16)]
        t = jnp.minimum(iv, 1)
        cnt = (jnp.take_along_axis(t, pa, axis=0)
               + jnp.take_along_axis(t, pb, axis=0)
               + jnp.take_along_axis(t, pc, axis=0))
        inv_d = jnp.where(cnt <= 1, 1.0,
                          jnp.where(cnt == 2, 0.5, 1.0 / 3.0))
        spb = jnp.take_along_axis(iv, p15, axis=0)
        sp_ = jnp.clip(spb - 1, 1, MHD)
        inv_sp = jnp.where(
            sp_ == 1, 1.0,
            jnp.where(sp_ == 2, 0.5,
                      jnp.where(sp_ == 3, 1.0 / 3.0,
                                jnp.where(sp_ == 4, 0.25, 0.2))))
        w = jnp.where(lane == 15, 1.0, inv_d * inv_sp)
        addr = (iv + offc) * HP
        acc = None
        for s in range(16):
            rowv = tbl_v[pl.ds(addr[s], 16)]
            rb = plsc.bitcast(rowv, jnp.bfloat16)
            ws = jnp.take_along_axis(w, lane * 0 + s, axis=0)
            wsb = plsc.pack(ws, ws, format=plsc.PackFormat.INTERLEAVED)
            c = rb * wsb
            acc = c if acc is None else acc + c
        plsc.store_scatter(r_v, [ioN + j], plsc.bitcast(acc, jnp.int32))
        return carry2

    lax.fori_loop(0, N, pair_body, 0, unroll=2)


def _sc_body(tcat_hbm, in_hbm, out_hbm, tbl_v, in0, in1, r0, r1,
             sem_t, si0, si1, so0, so1):
    wid = lax.axis_index("s") * NC + lax.axis_index("c")
    base = wid * ROWS_PER_W
    last = base + ROWS_PER_W - 1
    pltpu.async_copy(tcat_hbm, tbl_v, sem_t).wait()
    pltpu.async_copy(in_hbm.at[base], in0, si0)

    lane = lax.iota(jnp.int32, L)
    offc = lax.div(lane, EF) * NER
    pa = lax.div(lane, 3) * 3
    pb = jnp.minimum(pa + 1, 15)
    pc = jnp.minimum(pa + 2, 15)
    p15 = lane * 0 + 15
    ioN = lane * N
    consts = (lane, offc, pa, pb, pc, p15, ioN)

    def pair_body(k2, carry):
        row0 = base + 2 * k2
        row1 = row0 + 1
        nxt = jnp.minimum(row1 + 1, last)

        pltpu.make_async_copy(in_hbm.at[row0], in0, si0).wait()
        pltpu.async_copy(in_hbm.at[row1], in1, si1)

        @pl.when(k2 > 0)
        def _():
            pltpu.make_async_copy(r0, out_hbm.at[row0], so0).wait()

        _row_compute(in0, r0, tbl_v, *consts)
        pltpu.async_copy(r0, out_hbm.at[row0], so0)

        pltpu.make_async_copy(in_hbm.at[row1], in1, si1).wait()
        pltpu.async_copy(in_hbm.at[nxt], in0, si0)

        @pl.when(k2 > 0)
        def _():
            pltpu.make_async_copy(r1, out_hbm.at[row1], so1).wait()

        _row_compute(in1, r1, tbl_v, *consts)
        pltpu.async_copy(r1, out_hbm.at[row1], so1)
        return carry

    lax.fori_loop(0, ROWS_PER_W // 2, pair_body, 0)
    pltpu.make_async_copy(r0, out_hbm.at[last - 1], so0).wait()
    pltpu.make_async_copy(r1, out_hbm.at[last], so1).wait()
    pltpu.make_async_copy(in_hbm.at[last], in0, si0).wait()


_sc_gather = functools.partial(
    pl.kernel,
    out_type=jax.ShapeDtypeStruct((ROWS, HP * N), jnp.int32),
    mesh=plsc.VectorSubcoreMesh(core_axis_name="c", subcore_axis_name="s"),
    compiler_params=pltpu.CompilerParams(needs_layout_passes=False),
    scratch_types=[
        pltpu.VMEM((TBL_ROWS * HP,), jnp.int32),
        pltpu.VMEM((INW,), jnp.int32),
        pltpu.VMEM((INW,), jnp.int32),
        pltpu.VMEM((HP * N,), jnp.int32),
        pltpu.VMEM((HP * N,), jnp.int32),
        pltpu.SemaphoreType.DMA,
        pltpu.SemaphoreType.DMA,
        pltpu.SemaphoreType.DMA,
        pltpu.SemaphoreType.DMA,
        pltpu.SemaphoreType.DMA,
    ],
)(_sc_body)


def _add_body(ab_ref, r_ref, gt_ref, out_ref):
    h = pl.program_id(1)
    t = gt_ref[0, h]
    ab = ab_ref[0, 0]
    hp = lax.div(h, 2)
    sel = lax.rem(h, 2)
    w = r_ref[0, :, pl.ds(hp * N, N)]
    bits = jnp.where(sel == 0, w << 16,
                     w & jnp.int32(-65536))
    r = lax.bitcast_convert_type(bits, jnp.float32)
    rp = jnp.pad(r, ((1, 0), (1, 0)))
    i0 = lax.broadcasted_iota(jnp.int32, (N + 1, N + 1), 0)
    i1 = lax.broadcasted_iota(jnp.int32, (N + 1, N + 1), 1)
    edge_mask = jnp.logical_or(i0 == 0, i1 == 0)
    out_ref[0, 0] = ab + rp + jnp.where(edge_mask, t, 0.0)


def _add_bias(attn_bias, rpk, graph_token_weight):
    return pl.pallas_call(
        _add_body,
        grid=(B, H),
        in_specs=[
            pl.BlockSpec((1, 1, N + 1, N + 1), lambda b, h: (b, h, 0, 0)),
            pl.BlockSpec((1, N, HP * N), lambda b, h: (b, 0, 0)),
            pl.BlockSpec(memory_space=pltpu.SMEM),
        ],
        out_specs=pl.BlockSpec((1, 1, N + 1, N + 1), lambda b, h: (b, h, 0, 0)),
        out_shape=jax.ShapeDtypeStruct((B, H, N + 1, N + 1), jnp.float32),
        compiler_params=pltpu.CompilerParams(
            dimension_semantics=("arbitrary", "arbitrary")),
    )(attn_bias, rpk, graph_token_weight)


def kernel(attn_bias, spatial_pos, edge_input, attn_edge_type,
           edge_weight, spatial_weight, graph_token_weight, edge_dis_weight):
    del attn_edge_type
    w3 = edge_dis_weight.reshape(-1, H, H)[:MHD]
    tcat = _prep_table(edge_weight, w3, spatial_weight)
    packed = lax.bitcast_convert_type(
        tcat.astype(jnp.bfloat16).reshape(TBL_ROWS, HP, 2), jnp.int32)
    tflat = packed.reshape(TBL_ROWS * HP)
    in_t = jnp.concatenate(
        [edge_input.reshape(ROWS, N, MHD * EF),
         spatial_pos.reshape(ROWS, N, 1)], axis=2).reshape(ROWS, INW)
    r = _sc_gather(tflat, in_t)
    rpk = r.reshape(B, N, HP * N)
    return _add_bias(attn_bias, rpk, graph_token_weight)


# speedup vs baseline: 1.0033x; 1.0033x over previous
import functools

import jax
import jax.numpy as jnp
from jax import lax
from jax.experimental import pallas as pl
from jax.experimental.pallas import tpu as pltpu
from jax.experimental.pallas import tpu_sc as plsc

B, N, H = 16, 128, 32
MHD, EF = 5, 3
NER = 513
NSP = 512
TBL_ROWS = MHD * NER + NSP
HP = H // 2
NC, NS, L = 2, 16, 16
NW = NC * NS
ROWS = B * N
ROWS_PER_W = ROWS // NW
INW = N * 16


def _prep_body(ew_ref, w3_ref, sw_ref, out_ref):
    ew = ew_ref[...]
    parts = [jnp.dot(ew, w3_ref[m], preferred_element_type=jnp.float32)
             for m in range(MHD)]
    parts.append(sw_ref[...])
    out_ref[...] = jnp.concatenate(parts, axis=0)


def _prep_table(edge_weight, w3, spatial_weight):
    return pl.pallas_call(
        _prep_body,
        out_shape=jax.ShapeDtypeStruct((TBL_ROWS, H), jnp.float32),
    )(edge_weight, w3, spatial_weight)


def _row_compute(in_v, r_v, tbl_v, lane, offc, pa, pb, pc, p15, ioN):
    def pair_body(j, carry2):
        iv = in_v[pl.ds(j * 16, 16)]
        t = jnp.minimum(iv, 1)
        cnt = (jnp.take_along_axis(t, pa, axis=0)
               + jnp.take_along_axis(t, pb, axis=0)
               + jnp.take_along_axis(t, pc, axis=0))
        inv_d = jnp.where(cnt <= 1, 1.0,
                          jnp.where(cnt == 2, 0.5, 1.0 / 3.0))
        spb = jnp.take_along_axis(iv, p15, axis=0)
        sp_ = jnp.clip(spb - 1, 1, MHD)
        inv_sp = jnp.where(
            sp_ == 1, 1.0,
            jnp.where(sp_ == 2, 0.5,
                      jnp.where(sp_ == 3, 1.0 / 3.0,
                                jnp.where(sp_ == 4, 0.25, 0.2))))
        w = jnp.where(lane == 15, 1.0, inv_d * inv_sp)
        addr = (iv + offc) * HP
        acc = None
        for s in range(16):
            rowv = tbl_v[pl.ds(addr[s], 16)]
            rb = plsc.bitcast(rowv, jnp.bfloat16)
            ws = jnp.take_along_axis(w, lane * 0 + s, axis=0)
            wsb = plsc.pack(ws, ws, format=plsc.PackFormat.INTERLEAVED)
            c = rb * wsb
            acc = c if acc is None else acc + c
        plsc.store_scatter(r_v, [ioN + j], plsc.bitcast(acc, jnp.int32))
        return carry2

    lax.fori_loop(0, N, pair_body, 0)


def _sc_body(tcat_hbm, in_hbm, out_hbm, tbl_v, in0, in1, r0, r1,
             sem_t, si0, si1, so0, so1):
    wid = lax.axis_index("s") * NC + lax.axis_index("c")
    base = wid * ROWS_PER_W
    last = base + ROWS_PER_W - 1
    pltpu.async_copy(tcat_hbm, tbl_v, sem_t).wait()
    pltpu.async_copy(in_hbm.at[base], in0, si0)

    lane = lax.iota(jnp.int32, L)
    offc = lax.div(lane, EF) * NER
    pa = lax.div(lane, 3) * 3
    pb = jnp.minimum(pa + 1, 15)
    pc = jnp.minimum(pa + 2, 15)
    p15 = lane * 0 + 15
    ioN = lane * N
    consts = (lane, offc, pa, pb, pc, p15, ioN)

    def pair_body(k2, carry):
        row0 = base + 2 * k2
        row1 = row0 + 1
        nxt = jnp.minimum(row1 + 1, last)

        pltpu.make_async_copy(in_hbm.at[row0], in0, si0).wait()
        pltpu.async_copy(in_hbm.at[row1], in1, si1)

        @pl.when(k2 > 0)
        def _():
            pltpu.make_async_copy(r0, out_hbm.at[row0], so0).wait()

        _row_compute(in0, r0, tbl_v, *consts)
        pltpu.async_copy(r0, out_hbm.at[row0], so0)

        pltpu.make_async_copy(in_hbm.at[row1], in1, si1).wait()
        pltpu.async_copy(in_hbm.at[nxt], in0, si0)

        @pl.when(k2 > 0)
        def _():
            pltpu.make_async_copy(r1, out_hbm.at[row1], so1).wait()

        _row_compute(in1, r1, tbl_v, *consts)
        pltpu.async_copy(r1, out_hbm.at[row1], so1)
        return carry

    lax.fori_loop(0, ROWS_PER_W // 2, pair_body, 0)
    pltpu.make_async_copy(r0, out_hbm.at[last - 1], so0).wait()
    pltpu.make_async_copy(r1, out_hbm.at[last], so1).wait()
    pltpu.make_async_copy(in_hbm.at[last], in0, si0).wait()


_sc_gather = functools.partial(
    pl.kernel,
    out_type=jax.ShapeDtypeStruct((ROWS, HP * N), jnp.int32),
    mesh=plsc.VectorSubcoreMesh(core_axis_name="c", subcore_axis_name="s"),
    compiler_params=pltpu.CompilerParams(needs_layout_passes=False),
    scratch_types=[
        pltpu.VMEM((TBL_ROWS * HP,), jnp.int32),
        pltpu.VMEM((INW,), jnp.int32),
        pltpu.VMEM((INW,), jnp.int32),
        pltpu.VMEM((HP * N,), jnp.int32),
        pltpu.VMEM((HP * N,), jnp.int32),
        pltpu.SemaphoreType.DMA,
        pltpu.SemaphoreType.DMA,
        pltpu.SemaphoreType.DMA,
        pltpu.SemaphoreType.DMA,
        pltpu.SemaphoreType.DMA,
    ],
)(_sc_body)


def _add_body(ab_ref, r_ref, gt_ref, out_ref):
    h = pl.program_id(1)
    t = gt_ref[0, h]
    ab = ab_ref[0, 0]
    hp = lax.div(h, 2)
    sel = lax.rem(h, 2)
    w = r_ref[0, :, pl.ds(hp * N, N)]
    bits = jnp.where(sel == 0, w << 16,
                     w & jnp.int32(-65536))
    r = lax.bitcast_convert_type(bits, jnp.float32)
    rp = jnp.pad(r, ((1, 0), (1, 0)))
    i0 = lax.broadcasted_iota(jnp.int32, (N + 1, N + 1), 0)
    i1 = lax.broadcasted_iota(jnp.int32, (N + 1, N + 1), 1)
    edge_mask = jnp.logical_or(i0 == 0, i1 == 0)
    out_ref[0, 0] = ab + rp + jnp.where(edge_mask, t, 0.0)


def _add_bias(attn_bias, rpk, graph_token_weight):
    return pl.pallas_call(
        _add_body,
        grid=(B, H),
        in_specs=[
            pl.BlockSpec((1, 1, N + 1, N + 1), lambda b, h: (b, h, 0, 0)),
            pl.BlockSpec((1, N, HP * N), lambda b, h: (b, 0, 0)),
            pl.BlockSpec(memory_space=pltpu.SMEM),
        ],
        out_specs=pl.BlockSpec((1, 1, N + 1, N + 1), lambda b, h: (b, h, 0, 0)),
        out_shape=jax.ShapeDtypeStruct((B, H, N + 1, N + 1), jnp.float32),
        compiler_params=pltpu.CompilerParams(
            dimension_semantics=("arbitrary", "arbitrary")),
    )(attn_bias, rpk, graph_token_weight)


def kernel(attn_bias, spatial_pos, edge_input, attn_edge_type,
           edge_weight, spatial_weight, graph_token_weight, edge_dis_weight):
    del attn_edge_type
    w3 = edge_dis_weight.reshape(-1, H, H)[:MHD]
    tcat = _prep_table(edge_weight, w3, spatial_weight)
    packed = lax.bitcast_convert_type(
        tcat.astype(jnp.bfloat16).reshape(TBL_ROWS, HP, 2), jnp.int32)
    tflat = packed.reshape(TBL_ROWS * HP)
    in_t = jnp.concatenate(
        [edge_input.reshape(ROWS, N, MHD * EF),
         spatial_pos.reshape(ROWS, N, 1)], axis=2).reshape(ROWS, INW)
    r = _sc_gather(tflat, in_t)
    rpk = r.reshape(B, N, HP * N)
    return _add_bias(attn_bias, rpk, graph_token_weight)
